# baseline (device time: 162300 ns/iter reference)
import os

import jax
import jax.numpy as jnp
from jax import lax
from jax.experimental import pallas as pl
from jax.experimental.pallas import tpu as pltpu

N_DEV = 4
try:
    with open(os.path.join(os.path.dirname(__file__), "kdiag.txt")) as _f:
        _DIAG = _f.read().strip()
except OSError:
    _DIAG = ""


def kernel(x, w_mat):
    m, k_shard = x.shape
    _, n = w_mat.shape
    m_out = m // N_DEV

    n_half = n // 2

    def body(x_ref, w_ref, out_ref,
             send_bufs, recv_bufs, send_sems, recv_sems,
             my_amax, peer_amax, amax_send_sems, amax_recv_sems):
        d = lax.axis_index("i")
        left = (d - 1) % N_DEV
        right = (d + 1) % N_DEV

        peer_amax[...] = jnp.zeros((N_DEV, 8, 128), jnp.float32)

        barrier_sem = pltpu.get_barrier_semaphore()
        for nbr in [left, right]:
            pl.semaphore_signal(
                barrier_sem, inc=1,
                device_id=(nbr,), device_id_type=pl.DeviceIdType.MESH,
            )
        pl.semaphore_wait(barrier_sem, 2)

        m_q = m_out // 2

        def qgemm(idx, dirn, s):
            if _DIAG == "comm":
                return jnp.zeros((m_q, n_half), jnp.float32)
            xs = x_ref[pl.ds(idx * m_out + s * m_q, m_q), :].astype(jnp.bfloat16)
            ws = w_ref[:, pl.ds(dirn * n_half, n_half)]
            return jnp.dot(xs, ws, preferred_element_type=jnp.float32)

        SUBS = [(0, 0), (1, 0), (0, 1), (1, 1)]

        def mk(h, dirn, s):
            tgt = right if dirn == 0 else left
            return pltpu.make_async_remote_copy(
                src_ref=send_bufs.at[dirn, s],
                dst_ref=recv_bufs.at[h, dirn, s],
                send_sem=send_sems.at[h, dirn, s],
                recv_sem=recv_sems.at[h, dirn, s],
                device_id=(tgt,),
                device_id_type=pl.DeviceIdType.MESH,
            )

        if _DIAG == "bw":
            send_bufs[0, 0] = jnp.zeros((m_out // 2, n_half), jnp.bfloat16)
            probes = []
            for h in range(N_DEV - 1):
                for dirn in range(2):
                    for s in range(2):
                        r = pltpu.make_async_remote_copy(
                            src_ref=send_bufs.at[0, 0],
                            dst_ref=recv_bufs.at[h, dirn, s],
                            send_sem=send_sems.at[h, dirn, s],
                            recv_sem=recv_sems.at[h, dirn, s],
                            device_id=(right,),
                            device_id_type=pl.DeviceIdType.MESH,
                        )
                        r.start()
                        probes.append(r)
            for r in probes:
                r.wait_recv()
            for r in probes:
                r.wait_send()
            out_ref[...] = jnp.zeros((m_out, n), jnp.float32)
            return

        start_idx = {0: (d + 3) % N_DEV, 1: (d + 1) % N_DEV}
        inflight = {}
        for dirn, s in SUBS:
            g0 = qgemm(start_idx[dirn], dirn, s)
            send_bufs[dirn, s] = g0.astype(jnp.bfloat16)
            if _DIAG != "compute":
                r = mk(0, dirn, s)
                r.start()
                inflight[(dirn, s)] = r

        local_amax = jnp.float32(0.0)
        for h in range(N_DEV - 1):
            nidx = {0: (d + 2 - h) % N_DEV, 1: (d + 2 + h) % N_DEV}
            g = {}
            for dirn, s in SUBS:
                g[(dirn, s)] = qgemm(nidx[dirn], dirn, s)
            for dirn, s in SUBS:
                if _DIAG != "compute":
                    rd = inflight[(dirn, s)]
                    rd.wait_recv()
                    rd.wait_send()
                val = g[(dirn, s)] + recv_bufs[h, dirn, s].astype(jnp.float32)
                if h < N_DEV - 2:
                    send_bufs[dirn, s] = val.astype(jnp.bfloat16)
                    if _DIAG != "compute":
                        nxt = mk(h + 1, dirn, s)
                        nxt.start()
                        inflight[(dirn, s)] = nxt
                else:
                    out_ref[pl.ds(s * m_q, m_q), pl.ds(dirn * n_half, n_half)] = val
                    local_amax = jnp.maximum(local_amax, jnp.max(jnp.abs(val)))

        my_amax[...] = jnp.full((8, 128), local_amax, jnp.float32)
        peers = [right, (d + 2) % N_DEV, left] if _DIAG != "compute" else []
        rdmas = []
        for p in peers:
            r = pltpu.make_async_remote_copy(
                src_ref=my_amax,
                dst_ref=peer_amax.at[d],
                send_sem=amax_send_sems.at[p],
                recv_sem=amax_recv_sems.at[d],
                device_id=(p,),
                device_id_type=pl.DeviceIdType.MESH,
            )
            r.start()
            rdmas.append(r)
        for r in rdmas:
            r.wait_send()
        for p in peers:
            recv = pltpu.make_async_remote_copy(
                src_ref=my_amax,
                dst_ref=peer_amax.at[p],
                send_sem=amax_send_sems.at[p],
                recv_sem=amax_recv_sems.at[p],
                device_id=(p,),
                device_id_type=pl.DeviceIdType.MESH,
            )
            recv.wait_recv()
        global_amax = jnp.maximum(local_amax, jnp.max(peer_amax[...]))

        scale = global_amax / 127.0
        yq = out_ref[...]
        q = jnp.clip(jnp.round(yq / scale), -127.0, 127.0)
        out_ref[...] = q * scale

    return pl.pallas_call(
        body,
        out_shape=jax.ShapeDtypeStruct((m_out, n), jnp.float32),
        in_specs=[
            pl.BlockSpec(memory_space=pltpu.VMEM),
            pl.BlockSpec(memory_space=pltpu.VMEM),
        ],
        out_specs=pl.BlockSpec(memory_space=pltpu.VMEM),
        scratch_shapes=[
            pltpu.VMEM((2, 2, m_out // 2, n // 2), jnp.bfloat16),
            pltpu.VMEM((N_DEV - 1, 2, 2, m_out // 2, n // 2), jnp.bfloat16),
            pltpu.SemaphoreType.DMA((N_DEV - 1, 2, 2)),
            pltpu.SemaphoreType.DMA((N_DEV - 1, 2, 2)),
            pltpu.VMEM((8, 128), jnp.float32),
            pltpu.VMEM((N_DEV, 8, 128), jnp.float32),
            pltpu.SemaphoreType.DMA((N_DEV,)),
            pltpu.SemaphoreType.DMA((N_DEV,)),
        ],
        compiler_params=pltpu.CompilerParams(
            collective_id=0, vmem_limit_bytes=100 * 1024 * 1024
        ),
    )(x, w_mat.astype(jnp.bfloat16))


# device time: 74239 ns/iter; 2.1862x vs baseline; 2.1862x over previous
import os

import jax
import jax.numpy as jnp
from jax import lax
from jax.experimental import pallas as pl
from jax.experimental.pallas import tpu as pltpu

N_DEV = 4
try:
    with open(os.path.join(os.path.dirname(__file__), "kdiag.txt")) as _f:
        _DIAG = _f.read().strip()
except OSError:
    _DIAG = ""


def kernel(x, w_mat):
    m, k_shard = x.shape
    _, n = w_mat.shape
    m_out = m // N_DEV

    n_half = n // 2

    def body(x_ref, w_ref, out_ref,
             send_bufs, recv_bufs, send_sems, recv_sems,
             my_amax, peer_amax, amax_send_sems, amax_recv_sems):
        d = lax.axis_index("i")
        left = (d - 1) % N_DEV
        right = (d + 1) % N_DEV

        peer_amax[...] = jnp.zeros((N_DEV, 8, 128), jnp.float32)

        barrier_sem = pltpu.get_barrier_semaphore()
        for nbr in [left, right]:
            pl.semaphore_signal(
                barrier_sem, inc=1,
                device_id=(nbr,), device_id_type=pl.DeviceIdType.MESH,
            )
        pl.semaphore_wait(barrier_sem, 2)

        m_q = m_out // 2

        def qgemm(idx, dirn, s):
            if _DIAG == "comm":
                return jnp.zeros((m_q, n_half), jnp.float32)
            xs = x_ref[pl.ds(idx * m_out + s * m_q, m_q), :].astype(jnp.bfloat16)
            ws = w_ref[:, pl.ds(dirn * n_half, n_half)]
            return jnp.dot(xs, ws, preferred_element_type=jnp.float32)

        SUBS = [(0, 0), (1, 0), (0, 1), (1, 1)]

        def mk(h, dirn, s):
            tgt = right if dirn == 0 else left
            return pltpu.make_async_remote_copy(
                src_ref=send_bufs.at[dirn, s],
                dst_ref=recv_bufs.at[h, dirn, s],
                send_sem=send_sems.at[h, dirn, s],
                recv_sem=recv_sems.at[h, dirn, s],
                device_id=(tgt,),
                device_id_type=pl.DeviceIdType.MESH,
            )

        n_wire = n_half + 128

        def pack(val, dirn, s):
            row_amax = jnp.max(jnp.abs(val), axis=1, keepdims=True)
            e = jnp.ceil(8.0 * jnp.log2(jnp.maximum(row_amax, 1e-30) / 127.0))
            e = jnp.clip(e, -126.0, 126.0)
            inv = jnp.exp2(e * -0.125)
            q = jnp.clip(jnp.round(val * inv), -127.0, 127.0)
            send_bufs[dirn, s, :, pl.ds(0, n_half)] = q.astype(jnp.int8)
            send_bufs[dirn, s, :, pl.ds(n_half, 128)] = jnp.broadcast_to(
                e, (val.shape[0], 128)
            ).astype(jnp.int8)

        def unpack(h, dirn, s):
            raw = recv_bufs[h, dirn, s, :, pl.ds(0, n_half)].astype(jnp.float32)
            e = recv_bufs[h, dirn, s, :, pl.ds(n_half, 1)].astype(jnp.float32)
            return raw * jnp.exp2(e * 0.125)

        if _DIAG == "bw":
            send_bufs[0, 0] = jnp.zeros((m_out // 2, n_wire), jnp.int8)
            probes = []
            for h in range(N_DEV - 1):
                for dirn in range(2):
                    for s in range(2):
                        r = pltpu.make_async_remote_copy(
                            src_ref=send_bufs.at[0, 0],
                            dst_ref=recv_bufs.at[h, dirn, s],
                            send_sem=send_sems.at[h, dirn, s],
                            recv_sem=recv_sems.at[h, dirn, s],
                            device_id=(right,),
                            device_id_type=pl.DeviceIdType.MESH,
                        )
                        r.start()
                        probes.append(r)
            for r in probes:
                r.wait_recv()
            for r in probes:
                r.wait_send()
            out_ref[...] = jnp.zeros((m_out, n), jnp.float32)
            return

        start_idx = {0: (d + 3) % N_DEV, 1: (d + 1) % N_DEV}
        inflight = {}
        for dirn, s in SUBS:
            g0 = qgemm(start_idx[dirn], dirn, s)
            pack(g0, dirn, s)
            if _DIAG != "compute":
                r = mk(0, dirn, s)
                r.start()
                inflight[(dirn, s)] = r

        local_amax = jnp.float32(0.0)
        for h in range(N_DEV - 1):
            nidx = {0: (d + 2 - h) % N_DEV, 1: (d + 2 + h) % N_DEV}
            g = {}
            for dirn, s in SUBS:
                g[(dirn, s)] = qgemm(nidx[dirn], dirn, s)
            for dirn, s in SUBS:
                if _DIAG != "compute":
                    rd = inflight[(dirn, s)]
                    rd.wait_recv()
                    rd.wait_send()
                val = g[(dirn, s)] + unpack(h, dirn, s)
                if h < N_DEV - 2:
                    pack(val, dirn, s)
                    if _DIAG != "compute":
                        nxt = mk(h + 1, dirn, s)
                        nxt.start()
                        inflight[(dirn, s)] = nxt
                else:
                    out_ref[pl.ds(s * m_q, m_q), pl.ds(dirn * n_half, n_half)] = val
                    local_amax = jnp.maximum(local_amax, jnp.max(jnp.abs(val)))

        my_amax[...] = jnp.full((8, 128), local_amax, jnp.float32)
        peers = [right, (d + 2) % N_DEV, left] if _DIAG != "compute" else []
        rdmas = []
        for p in peers:
            r = pltpu.make_async_remote_copy(
                src_ref=my_amax,
                dst_ref=peer_amax.at[d],
                send_sem=amax_send_sems.at[p],
                recv_sem=amax_recv_sems.at[d],
                device_id=(p,),
                device_id_type=pl.DeviceIdType.MESH,
            )
            r.start()
            rdmas.append(r)
        for r in rdmas:
            r.wait_send()
        for p in peers:
            recv = pltpu.make_async_remote_copy(
                src_ref=my_amax,
                dst_ref=peer_amax.at[p],
                send_sem=amax_send_sems.at[p],
                recv_sem=amax_recv_sems.at[p],
                device_id=(p,),
                device_id_type=pl.DeviceIdType.MESH,
            )
            recv.wait_recv()
        global_amax = jnp.maximum(local_amax, jnp.max(peer_amax[...]))

        scale = global_amax / 127.0
        yq = out_ref[...]
        q = jnp.clip(jnp.round(yq / scale), -127.0, 127.0)
        out_ref[...] = q * scale

    return pl.pallas_call(
        body,
        out_shape=jax.ShapeDtypeStruct((m_out, n), jnp.float32),
        in_specs=[
            pl.BlockSpec(memory_space=pltpu.VMEM),
            pl.BlockSpec(memory_space=pltpu.VMEM),
        ],
        out_specs=pl.BlockSpec(memory_space=pltpu.VMEM),
        scratch_shapes=[
            pltpu.VMEM((2, 2, m_out // 2, n // 2 + 128), jnp.int8),
            pltpu.VMEM((N_DEV - 1, 2, 2, m_out // 2, n // 2 + 128), jnp.int8),
            pltpu.SemaphoreType.DMA((N_DEV - 1, 2, 2)),
            pltpu.SemaphoreType.DMA((N_DEV - 1, 2, 2)),
            pltpu.VMEM((8, 128), jnp.float32),
            pltpu.VMEM((N_DEV, 8, 128), jnp.float32),
            pltpu.SemaphoreType.DMA((N_DEV,)),
            pltpu.SemaphoreType.DMA((N_DEV,)),
        ],
        compiler_params=pltpu.CompilerParams(
            collective_id=0, vmem_limit_bytes=100 * 1024 * 1024
        ),
    )(x, w_mat.astype(jnp.bfloat16))


# device time: 72686 ns/iter; 2.2329x vs baseline; 1.0214x over previous
import os

import jax
import jax.numpy as jnp
from jax import lax
from jax.experimental import pallas as pl
from jax.experimental.pallas import tpu as pltpu

N_DEV = 4
try:
    with open(os.path.join(os.path.dirname(__file__), "kdiag.txt")) as _f:
        _DIAG = _f.read().strip()
except OSError:
    _DIAG = ""


def kernel(x, w_mat):
    m, k_shard = x.shape
    _, n = w_mat.shape
    m_out = m // N_DEV

    n_half = n // 2

    def body(x_ref, w_ref, out_ref,
             send_bufs, recv_bufs, send_sems, recv_sems,
             my_amax, peer_amax, amax_send_sems, amax_recv_sems):
        d = lax.axis_index("i")
        left = (d - 1) % N_DEV
        right = (d + 1) % N_DEV

        peer_amax[...] = jnp.zeros((N_DEV, 8, 128), jnp.float32)

        barrier_sem = pltpu.get_barrier_semaphore()
        for nbr in [left, right]:
            pl.semaphore_signal(
                barrier_sem, inc=1,
                device_id=(nbr,), device_id_type=pl.DeviceIdType.MESH,
            )
        pl.semaphore_wait(barrier_sem, 2)

        m_q = m_out // 2

        def qgemm(idx, dirn, s):
            if _DIAG == "comm":
                return jnp.zeros((m_q, n_half), jnp.float32)
            xs = x_ref[pl.ds(idx * m_out + s * m_q, m_q), :].astype(jnp.bfloat16)
            ws = w_ref[:, pl.ds(dirn * n_half, n_half)]
            return jnp.dot(xs, ws, preferred_element_type=jnp.float32)

        SUBS = [(0, 0), (1, 0), (0, 1), (1, 1)]

        def mk(h, dirn, s):
            tgt = right if dirn == 0 else left
            return pltpu.make_async_remote_copy(
                src_ref=send_bufs.at[dirn, s],
                dst_ref=recv_bufs.at[h, dirn, s],
                send_sem=send_sems.at[h, dirn, s],
                recv_sem=recv_sems.at[h, dirn, s],
                device_id=(tgt,),
                device_id_type=pl.DeviceIdType.MESH,
            )


        def pack(val, dirn, s):
            col_amax = jnp.max(jnp.abs(val), axis=0, keepdims=True)
            e = jnp.ceil(8.0 * jnp.log2(jnp.maximum(col_amax, 1e-30) / 127.0))
            e = jnp.clip(e, -126.0, 126.0)
            inv = jnp.exp2(e * -0.125)
            q = jnp.clip(jnp.round(val * inv), -127.0, 127.0)
            send_bufs[dirn, s, pl.ds(0, m_q), :] = q.astype(jnp.int8)
            send_bufs[dirn, s, pl.ds(m_q, 32), :] = jnp.broadcast_to(
                e, (32, n_half)
            ).astype(jnp.int8)

        def unpack(h, dirn, s):
            raw = recv_bufs[h, dirn, s, pl.ds(0, m_q), :].astype(jnp.float32)
            e = recv_bufs[h, dirn, s, pl.ds(m_q, 1), :].astype(jnp.float32)
            return raw * jnp.exp2(e * 0.125)

        if _DIAG == "bw":
            send_bufs[0, 0] = jnp.zeros((m_out // 2 + 32, n_half), jnp.int8)
            probes = []
            for h in range(N_DEV - 1):
                for dirn in range(2):
                    for s in range(2):
                        r = pltpu.make_async_remote_copy(
                            src_ref=send_bufs.at[0, 0],
                            dst_ref=recv_bufs.at[h, dirn, s],
                            send_sem=send_sems.at[h, dirn, s],
                            recv_sem=recv_sems.at[h, dirn, s],
                            device_id=(right,),
                            device_id_type=pl.DeviceIdType.MESH,
                        )
                        r.start()
                        probes.append(r)
            for r in probes:
                r.wait_recv()
            for r in probes:
                r.wait_send()
            out_ref[...] = jnp.zeros((m_out, n), jnp.float32)
            return

        start_idx = {0: (d + 3) % N_DEV, 1: (d + 1) % N_DEV}
        inflight = {}
        for dirn, s in SUBS:
            g0 = qgemm(start_idx[dirn], dirn, s)
            pack(g0, dirn, s)
            if _DIAG != "compute":
                r = mk(0, dirn, s)
                r.start()
                inflight[(dirn, s)] = r

        local_amax = jnp.float32(0.0)
        for h in range(N_DEV - 1):
            nidx = {0: (d + 2 - h) % N_DEV, 1: (d + 2 + h) % N_DEV}
            g = {}
            for dirn, s in SUBS:
                g[(dirn, s)] = qgemm(nidx[dirn], dirn, s)
            for dirn, s in SUBS:
                if _DIAG != "compute":
                    rd = inflight[(dirn, s)]
                    rd.wait_recv()
                    rd.wait_send()
                val = g[(dirn, s)] + unpack(h, dirn, s)
                if h < N_DEV - 2:
                    pack(val, dirn, s)
                    if _DIAG != "compute":
                        nxt = mk(h + 1, dirn, s)
                        nxt.start()
                        inflight[(dirn, s)] = nxt
                else:
                    out_ref[pl.ds(s * m_q, m_q), pl.ds(dirn * n_half, n_half)] = val
                    local_amax = jnp.maximum(local_amax, jnp.max(jnp.abs(val)))

        my_amax[...] = jnp.full((8, 128), local_amax, jnp.float32)
        peers = [right, (d + 2) % N_DEV, left] if _DIAG != "compute" else []
        rdmas = []
        for p in peers:
            r = pltpu.make_async_remote_copy(
                src_ref=my_amax,
                dst_ref=peer_amax.at[d],
                send_sem=amax_send_sems.at[p],
                recv_sem=amax_recv_sems.at[d],
                device_id=(p,),
                device_id_type=pl.DeviceIdType.MESH,
            )
            r.start()
            rdmas.append(r)
        for r in rdmas:
            r.wait_send()
        for p in peers:
            recv = pltpu.make_async_remote_copy(
                src_ref=my_amax,
                dst_ref=peer_amax.at[p],
                send_sem=amax_send_sems.at[p],
                recv_sem=amax_recv_sems.at[p],
                device_id=(p,),
                device_id_type=pl.DeviceIdType.MESH,
            )
            recv.wait_recv()
        global_amax = jnp.maximum(local_amax, jnp.max(peer_amax[...]))

        scale = global_amax / 127.0
        yq = out_ref[...]
        q = jnp.clip(jnp.round(yq / scale), -127.0, 127.0)
        out_ref[...] = q * scale

    return pl.pallas_call(
        body,
        out_shape=jax.ShapeDtypeStruct((m_out, n), jnp.float32),
        in_specs=[
            pl.BlockSpec(memory_space=pltpu.VMEM),
            pl.BlockSpec(memory_space=pltpu.VMEM),
        ],
        out_specs=pl.BlockSpec(memory_space=pltpu.VMEM),
        scratch_shapes=[
            pltpu.VMEM((2, 2, m_out // 2 + 32, n // 2), jnp.int8),
            pltpu.VMEM((N_DEV - 1, 2, 2, m_out // 2 + 32, n // 2), jnp.int8),
            pltpu.SemaphoreType.DMA((N_DEV - 1, 2, 2)),
            pltpu.SemaphoreType.DMA((N_DEV - 1, 2, 2)),
            pltpu.VMEM((8, 128), jnp.float32),
            pltpu.VMEM((N_DEV, 8, 128), jnp.float32),
            pltpu.SemaphoreType.DMA((N_DEV,)),
            pltpu.SemaphoreType.DMA((N_DEV,)),
        ],
        compiler_params=pltpu.CompilerParams(
            collective_id=0, vmem_limit_bytes=100 * 1024 * 1024
        ),
    )(x, w_mat.astype(jnp.bfloat16))
